# Initial kernel scaffold; baseline (speedup 1.0000x reference)
import jax
import jax.numpy as jnp
from jax.experimental import pallas as pl


def kernel(edge_features, neighbor_mask, W_lin, b_lin, ln1_g, ln1_b, W1, b1, W2, b2, ln2_g, ln2_b):
    raise NotImplementedError("write your pallas kernel here")


# speedup vs baseline: 2.0252x; 2.0252x over previous
import functools

import jax
import jax.numpy as jnp
from jax.experimental import pallas as pl


def _block_kernel(ef_ref, mask_ref, wlin_ref, blin_ref, g1_ref, b1_ref,
                  w1_ref, bf1_ref, w2_ref, bf2_ref, g2_ref, b2_ref,
                  node_out_ref, edge_out_ref, *, bn, k, d, h):
    x = ef_ref[...].reshape(bn * k, d)
    eh = jnp.dot(x, wlin_ref[...], preferred_element_type=jnp.float32)
    eh = eh + blin_ref[...]

    mu = jnp.mean(eh, axis=-1, keepdims=True)
    var = jnp.mean(jnp.square(eh - mu), axis=-1, keepdims=True)
    hn = (eh - mu) * jax.lax.rsqrt(var + 1e-5) * g1_ref[...] + b1_ref[...]

    f = jnp.dot(hn, w1_ref[...], preferred_element_type=jnp.float32)
    f = jax.nn.gelu(f + bf1_ref[...])
    f = jnp.dot(f, w2_ref[...], preferred_element_type=jnp.float32)
    eo = eh + f + bf2_ref[...]

    edge_out_ref[...] = eo.reshape(bn, k, h)

    m = mask_ref[...]
    agg = jnp.sum(eo.reshape(bn, k, h) * m[:, :, None], axis=1)

    mu2 = jnp.mean(agg, axis=-1, keepdims=True)
    var2 = jnp.mean(jnp.square(agg - mu2), axis=-1, keepdims=True)
    node_out_ref[...] = ((agg - mu2) * jax.lax.rsqrt(var2 + 1e-5)
                         * g2_ref[...] + b2_ref[...])


def kernel(edge_features, neighbor_mask, W_lin, b_lin, ln1_g, ln1_b,
           W1, b1, W2, b2, ln2_g, ln2_b):
    n, k, d = edge_features.shape
    h = W_lin.shape[1]
    bn = 80
    assert n % bn == 0
    grid = (n // bn,)

    mask_f = neighbor_mask.astype(jnp.float32)
    vec = lambda v: v.reshape(1, -1)

    row_spec = lambda shape: pl.BlockSpec(shape, lambda i: (i,) + (0,) * (len(shape) - 1))
    full_spec = lambda shape: pl.BlockSpec(shape, lambda i: (0,) * len(shape))

    node_out, edge_out = pl.pallas_call(
        functools.partial(_block_kernel, bn=bn, k=k, d=d, h=h),
        grid=grid,
        in_specs=[
            row_spec((bn, k, d)),
            row_spec((bn, k)),
            full_spec((d, h)),
            full_spec((1, h)),
            full_spec((1, h)),
            full_spec((1, h)),
            full_spec((h, h)),
            full_spec((1, h)),
            full_spec((h, h)),
            full_spec((1, h)),
            full_spec((1, h)),
            full_spec((1, h)),
        ],
        out_specs=[
            row_spec((bn, h)),
            row_spec((bn, k, h)),
        ],
        out_shape=[
            jax.ShapeDtypeStruct((n, h), jnp.float32),
            jax.ShapeDtypeStruct((n, k, h), jnp.float32),
        ],
    )(edge_features, mask_f, W_lin, vec(b_lin), vec(ln1_g), vec(ln1_b),
      W1, vec(b1), W2, vec(b2), vec(ln2_g), vec(ln2_b))

    return (node_out, edge_out)
